# baseline (speedup 1.0000x reference)
import functools

import jax
import jax.numpy as jnp
from jax import lax
from jax.experimental import pallas as pl
from jax.experimental.pallas import tpu as pltpu
from jax.experimental.pallas import tpu_sc as plsc


def _proj_body(x_ref, w1a_ref, w1b_ref, b1_ref, y1_ref, y2_ref):
    xb = x_ref[...]
    y1_ref[...] = (
        jnp.dot(xb, w1a_ref[...], preferred_element_type=jnp.float32)
        + b1_ref[...]
    )
    y2_ref[...] = jnp.dot(xb, w1b_ref[...], preferred_element_type=jnp.float32)


def _node_proj(x, W1a, W1b, b1):
    N, D = x.shape
    NB = 1000
    return pl.pallas_call(
        _proj_body,
        grid=(N // NB,),
        in_specs=[
            pl.BlockSpec((NB, D), lambda i: (i, 0)),
            pl.BlockSpec((D, D), lambda i: (0, 0)),
            pl.BlockSpec((D, D), lambda i: (0, 0)),
            pl.BlockSpec((1, D), lambda i: (0, 0)),
        ],
        out_specs=[
            pl.BlockSpec((NB, D), lambda i: (i, 0)),
            pl.BlockSpec((NB, D), lambda i: (i, 0)),
        ],
        out_shape=[jax.ShapeDtypeStruct((N, D), jnp.float32)] * 2,
    )(x, W1a, W1b, b1.reshape(1, D))


_NC = 2
_NS = 16
_NW = _NC * _NS
_K = 64


def _sc_gather(y1, y2, src2, dst2, e_pad):
    D = y1.shape[1]
    pw = e_pad // _NW
    nch = pw // _K
    mesh = plsc.VectorSubcoreMesh(
        core_axis_name="c", subcore_axis_name="s",
        num_cores=_NC, num_subcores=_NS,
    )

    @functools.partial(
        pl.kernel,
        mesh=mesh,
        out_type=jax.ShapeDtypeStruct((e_pad, D), jnp.float32),
        scratch_types=[
            pltpu.VMEM((nch, _K), jnp.int32),
            pltpu.VMEM((nch, _K), jnp.int32),
            [pltpu.VMEM((_K, D), jnp.float32)] * 2,
            [pltpu.VMEM((_K, D), jnp.float32)] * 2,
            [pltpu.VMEM((_K, D), jnp.float32)] * 2,
            [pltpu.SemaphoreType.DMA] * 2,
            [pltpu.SemaphoreType.DMA] * 2,
        ],
    )
    def gather_kernel(y1_hbm, y2_hbm, src_hbm, dst_hbm, g_hbm,
                      isa, ida, r1, r2, ro, sg, sw):
        wid = lax.axis_index("s") * _NC + lax.axis_index("c")
        base_row = wid * nch
        base = wid * pw

        pltpu.sync_copy(src_hbm.at[pl.ds(base_row, nch)], isa)
        pltpu.sync_copy(dst_hbm.at[pl.ds(base_row, nch)], ida)

        def issue_gathers(j, b):
            pltpu.async_copy(y1_hbm.at[isa.at[j]], r1[b], sg[b])
            pltpu.async_copy(y2_hbm.at[ida.at[j]], r2[b], sg[b])

        def wait_gathers(j, b):
            pltpu.make_async_copy(y1_hbm.at[isa.at[j]], r1[b], sg[b]).wait()
            pltpu.make_async_copy(y2_hbm.at[ida.at[j]], r2[b], sg[b]).wait()

        def out_slice(j):
            return g_hbm.at[pl.ds(base + j * _K, _K)]

        def add_rows(b):
            r1b, r2b, rob = r1[b], r2[b], ro[b]

            @plsc.parallel_loop(0, _K, unroll=4)
            def _(i):
                for t in range(D // 16):
                    sl = pl.ds(t * 16, 16)
                    rob[i, sl] = r1b[i, sl] + r2b[i, sl]

        def process(j, b, wait_write, issue_next):
            wait_gathers(j, b)
            if wait_write:
                pltpu.make_async_copy(ro[b], out_slice(j), sw[b]).wait()
            add_rows(b)
            if issue_next:
                issue_gathers(j + 2, b)
            pltpu.async_copy(ro[b], out_slice(j), sw[b])

        issue_gathers(0, 0)
        issue_gathers(1, 1)
        process(0, 0, wait_write=False, issue_next=True)
        process(1, 1, wait_write=False, issue_next=True)

        def step(j2, carry):
            process(2 * j2, 0, wait_write=True, issue_next=True)
            process(2 * j2 + 1, 1, wait_write=True, issue_next=True)
            return carry

        lax.fori_loop(1, nch // 2 - 1, step, 0)

        process(nch - 2, 0, wait_write=True, issue_next=False)
        process(nch - 1, 1, wait_write=True, issue_next=False)
        pltpu.make_async_copy(ro[0], out_slice(nch - 2), sw[0]).wait()
        pltpu.make_async_copy(ro[1], out_slice(nch - 1), sw[1]).wait()

    return gather_kernel(y1, y2, src2, dst2)


def _edge_body(g_ref, ea_ref, w2_ref, b2_ref, gm_ref, bt_ref, o_ref):
    t = g_ref[...]
    h = jnp.where(t > 0, t, jnp.exp(t) - 1.0)
    score = (
        jnp.dot(h, w2_ref[...], preferred_element_type=jnp.float32)
        + b2_ref[...]
    )
    r = ea_ref[...] + score
    mu = jnp.mean(r, axis=1, keepdims=True)
    c = r - mu
    var = jnp.mean(c * c, axis=1, keepdims=True)
    o_ref[...] = c * lax.rsqrt(var + 1e-5) * gm_ref[...] + bt_ref[...]


def _edge_stage(g, edge_attr, W2, b2, gamma, beta, E):
    D = edge_attr.shape[1]
    EB = 1600
    return pl.pallas_call(
        _edge_body,
        grid=(E // EB,),
        in_specs=[
            pl.BlockSpec((EB, D), lambda i: (i, 0)),
            pl.BlockSpec((EB, D), lambda i: (i, 0)),
            pl.BlockSpec((D, D), lambda i: (0, 0)),
            pl.BlockSpec((1, D), lambda i: (0, 0)),
            pl.BlockSpec((1, D), lambda i: (0, 0)),
            pl.BlockSpec((1, D), lambda i: (0, 0)),
        ],
        out_specs=pl.BlockSpec((EB, D), lambda i: (i, 0)),
        out_shape=jax.ShapeDtypeStruct((E, D), jnp.float32),
    )(g, edge_attr, W2, b2.reshape(1, D), gamma.reshape(1, D),
      beta.reshape(1, D))


def kernel(x, edge_index, edge_attr, W1, b1, W2, b2, gamma, beta):
    N, D = x.shape
    E = edge_index.shape[1]

    W1a = W1[:D]
    W1b = W1[D:]
    y1, y2 = _node_proj(x, W1a, W1b, b1)

    quantum = 2 * _NW * _K
    e_pad = ((E + quantum - 1) // quantum) * quantum
    src = edge_index[0].astype(jnp.int32)
    dst = edge_index[1].astype(jnp.int32)
    if e_pad != E:
        pad = e_pad - E
        src = jnp.concatenate([src, jnp.zeros((pad,), jnp.int32)])
        dst = jnp.concatenate([dst, jnp.zeros((pad,), jnp.int32)])
    src2 = src.reshape(e_pad // _K, _K)
    dst2 = dst.reshape(e_pad // _K, _K)

    g = _sc_gather(y1, y2, src2, dst2, e_pad)

    return _edge_stage(g, edge_attr, W2, b2, gamma, beta, E)


# speedup vs baseline: 1.0048x; 1.0048x over previous
import functools

import jax
import jax.numpy as jnp
from jax import lax
from jax.experimental import pallas as pl
from jax.experimental.pallas import tpu as pltpu
from jax.experimental.pallas import tpu_sc as plsc


def _proj_body(x_ref, w1a_ref, w1b_ref, b1_ref, y1_ref, y2_ref):
    xb = x_ref[...]
    y1_ref[...] = (
        jnp.dot(xb, w1a_ref[...], preferred_element_type=jnp.float32)
        + b1_ref[...]
    )
    y2_ref[...] = jnp.dot(xb, w1b_ref[...], preferred_element_type=jnp.float32)


def _node_proj(x, W1a, W1b, b1):
    N, D = x.shape
    NB = 1000
    return pl.pallas_call(
        _proj_body,
        grid=(N // NB,),
        in_specs=[
            pl.BlockSpec((NB, D), lambda i: (i, 0)),
            pl.BlockSpec((D, D), lambda i: (0, 0)),
            pl.BlockSpec((D, D), lambda i: (0, 0)),
            pl.BlockSpec((1, D), lambda i: (0, 0)),
        ],
        out_specs=[
            pl.BlockSpec((NB, D), lambda i: (i, 0)),
            pl.BlockSpec((NB, D), lambda i: (i, 0)),
        ],
        out_shape=[jax.ShapeDtypeStruct((N, D), jnp.float32)] * 2,
    )(x, W1a, W1b, b1.reshape(1, D))


_NC = 2
_NS = 16
_NW = _NC * _NS
_K = 64


def _sc_gather(y1, y2, src2, dst2, e_pad):
    D = y1.shape[1]
    nch = e_pad // (_NW * _K)
    pw = nch * _K
    mesh = plsc.VectorSubcoreMesh(core_axis_name="c", subcore_axis_name="s")

    @functools.partial(
        pl.kernel,
        mesh=mesh,
        out_type=jax.ShapeDtypeStruct((e_pad, D), jnp.float32),
        scratch_types=[
            pltpu.VMEM((nch, _K), jnp.int32),
            pltpu.VMEM((nch, _K), jnp.int32),
            [pltpu.VMEM((_K, D), jnp.float32)] * 2,
            [pltpu.VMEM((_K, D), jnp.float32)] * 2,
            [pltpu.VMEM((_K, D), jnp.float32)] * 2,
            [pltpu.SemaphoreType.DMA] * 2,
            [pltpu.SemaphoreType.DMA] * 2,
        ],
    )
    def gather_kernel(y1_hbm, y2_hbm, src_hbm, dst_hbm, g_hbm,
                      isa, ida, r1, r2, ro, sg, sw):
        wid = lax.axis_index("s") * _NC + lax.axis_index("c")
        base_row = wid * nch
        base = wid * pw

        pltpu.sync_copy(src_hbm.at[pl.ds(base_row, nch)], isa)
        pltpu.sync_copy(dst_hbm.at[pl.ds(base_row, nch)], ida)

        def issue_gathers(j, b):
            pltpu.async_copy(y1_hbm.at[isa.at[j]], r1[b], sg[b])
            pltpu.async_copy(y2_hbm.at[ida.at[j]], r2[b], sg[b])

        def wait_gathers(j, b):
            pltpu.make_async_copy(y1_hbm.at[isa.at[j]], r1[b], sg[b]).wait()
            pltpu.make_async_copy(y2_hbm.at[ida.at[j]], r2[b], sg[b]).wait()

        def out_slice(j):
            return g_hbm.at[pl.ds(base + j * _K, _K)]

        def add_rows(b):
            r1b, r2b, rob = r1[b], r2[b], ro[b]

            @plsc.parallel_loop(0, _K, unroll=4)
            def _(i):
                for t in range(D // 16):
                    sl = pl.ds(t * 16, 16)
                    rob[i, sl] = r1b[i, sl] + r2b[i, sl]

        def process(j, b, wait_write, issue_next):
            wait_gathers(j, b)
            if wait_write:
                pltpu.make_async_copy(ro[b], out_slice(j), sw[b]).wait()
            add_rows(b)
            if issue_next:
                issue_gathers(j + 2, b)
            pltpu.async_copy(ro[b], out_slice(j), sw[b])

        issue_gathers(0, 0)
        issue_gathers(1, 1)
        process(0, 0, wait_write=False, issue_next=True)
        process(1, 1, wait_write=False, issue_next=True)

        def step(j2, carry):
            process(2 * j2, 0, wait_write=True, issue_next=True)
            process(2 * j2 + 1, 1, wait_write=True, issue_next=True)
            return carry

        lax.fori_loop(1, nch // 2 - 1, step, 0)

        process(nch - 2, 0, wait_write=True, issue_next=False)
        process(nch - 1, 1, wait_write=True, issue_next=False)
        pltpu.make_async_copy(ro[0], out_slice(nch - 2), sw[0]).wait()
        pltpu.make_async_copy(ro[1], out_slice(nch - 1), sw[1]).wait()

    return gather_kernel(y1, y2, src2, dst2)


def _edge_body(g_ref, ea_ref, w2_ref, b2_ref, gm_ref, bt_ref, o_ref):
    t = g_ref[...]
    h = jnp.where(t > 0, t, jnp.exp(t) - 1.0)
    score = (
        jnp.dot(h, w2_ref[...], preferred_element_type=jnp.float32)
        + b2_ref[...]
    )
    r = ea_ref[...] + score
    mu = jnp.mean(r, axis=1, keepdims=True)
    c = r - mu
    var = jnp.mean(c * c, axis=1, keepdims=True)
    o_ref[...] = c * lax.rsqrt(var + 1e-5) * gm_ref[...] + bt_ref[...]


def _edge_stage(g, edge_attr, W2, b2, gamma, beta, E):
    D = edge_attr.shape[1]
    EB = 1600
    return pl.pallas_call(
        _edge_body,
        grid=(E // EB,),
        in_specs=[
            pl.BlockSpec((EB, D), lambda i: (i, 0)),
            pl.BlockSpec((EB, D), lambda i: (i, 0)),
            pl.BlockSpec((D, D), lambda i: (0, 0)),
            pl.BlockSpec((1, D), lambda i: (0, 0)),
            pl.BlockSpec((1, D), lambda i: (0, 0)),
            pl.BlockSpec((1, D), lambda i: (0, 0)),
        ],
        out_specs=pl.BlockSpec((EB, D), lambda i: (i, 0)),
        out_shape=jax.ShapeDtypeStruct((E, D), jnp.float32),
    )(g, edge_attr, W2, b2.reshape(1, D), gamma.reshape(1, D),
      beta.reshape(1, D))


def kernel(x, edge_index, edge_attr, W1, b1, W2, b2, gamma, beta):
    N, D = x.shape
    E = edge_index.shape[1]

    W1a = W1[:D]
    W1b = W1[D:]
    y1, y2 = _node_proj(x, W1a, W1b, b1)

    quantum = 2 * _NW * _K
    e_pad = ((E + quantum - 1) // quantum) * quantum
    src = edge_index[0].astype(jnp.int32)
    dst = edge_index[1].astype(jnp.int32)
    if e_pad != E:
        pad = e_pad - E
        src = jnp.concatenate([src, jnp.zeros((pad,), jnp.int32)])
        dst = jnp.concatenate([dst, jnp.zeros((pad,), jnp.int32)])
    src2 = src.reshape(e_pad // _K, _K)
    dst2 = dst.reshape(e_pad // _K, _K)

    g = _sc_gather(y1, y2, src2, dst2, e_pad)

    return _edge_stage(g, edge_attr, W2, b2, gamma, beta, E)
